# baseline (speedup 1.0000x reference)
import functools

import jax
import jax.numpy as jnp
from jax import lax
from jax.experimental import pallas as pl
from jax.experimental.pallas import tpu as pltpu
from jax.experimental.pallas import tpu_sc as plsc

_NUM_GENES = 16906
_EMB_DIM = 200
_NW = 32

_RSLAB = 8
_NR = _EMB_DIM // _RSLAB
_CBLK = 4224
_NC = 4
_MAIN = _NR * _NC
_TAIL_COL = _NC * _CBLK
_TAIL_W = _NUM_GENES - _TAIL_COL
_ITERS = 4


def _copy_body(src_hbm, out_hbm, buf0, buf1, s0, s1, s2, s3):
    wid = lax.axis_index("s") * 2 + lax.axis_index("c")
    bufs = (buf0, buf1)
    rsems = (s0, s1)
    wsems = (s2, s3)

    @pl.when(wid < _NR)
    def _():
        r = pl.multiple_of(wid * _RSLAB, 8)
        writes = {}
        for k in range(_NC):
            b = k % 2
            if k >= 2:
                writes[k - 2].wait()
            sl = (pl.ds(r, _RSLAB), pl.ds(k * _CBLK, _CBLK))
            pltpu.async_copy(src_hbm.at[sl], bufs[b], rsems[b]).wait()
            writes[k] = pltpu.async_copy(bufs[b], out_hbm.at[sl], wsems[b])
        writes[_NC - 2].wait()
        writes[_NC - 1].wait()


def kernel(x, gene_emb):
    del x
    src = gene_emb.T
    mesh = plsc.VectorSubcoreMesh(core_axis_name="c", subcore_axis_name="s")
    run = functools.partial(
        pl.kernel,
        mesh=mesh,
        out_type=jax.ShapeDtypeStruct((_EMB_DIM, _NUM_GENES), jnp.float32),
        scratch_types=[
            pltpu.VMEM((_RSLAB, _CBLK), jnp.float32),
            pltpu.VMEM((_RSLAB, _CBLK), jnp.float32),
            pltpu.SemaphoreType.DMA,
            pltpu.SemaphoreType.DMA,
            pltpu.SemaphoreType.DMA,
            pltpu.SemaphoreType.DMA,
        ],
    )(_copy_body)
    out = run(src)
    tail = lax.slice(src, (0, _TAIL_COL), (_EMB_DIM, _NUM_GENES))
    out = lax.dynamic_update_slice(out, tail, (0, _TAIL_COL))
    return out.T


# speedup vs baseline: 1.0531x; 1.0531x over previous
import functools

import jax
import jax.numpy as jnp
from jax import lax
from jax.experimental import pallas as pl
from jax.experimental.pallas import tpu as pltpu
from jax.experimental.pallas import tpu_sc as plsc

_NUM_GENES = 16906
_EMB_DIM = 200

_RSLAB = 8
_NSLABS = _EMB_DIM // _RSLAB
_SC0_SLABS = 13


def _copy_body(src_hbm, out_hbm, shared):
    c = lax.axis_index("c")
    s = lax.axis_index("s")
    slab = s + c * _SC0_SLABS
    n_mine = jnp.where(c == 0, _SC0_SLABS, _NSLABS - _SC0_SLABS)

    @pl.when(s < n_mine)
    def _():
        r = pl.multiple_of(slab * _RSLAB, 8)
        pltpu.sync_copy(src_hbm.at[pl.ds(r, _RSLAB), :], shared.at[s])
        pltpu.sync_copy(shared.at[s], out_hbm.at[pl.ds(r, _RSLAB), :])


def kernel(x, gene_emb):
    del x
    src = gene_emb.T
    mesh = plsc.VectorSubcoreMesh(core_axis_name="c", subcore_axis_name="s")
    run = functools.partial(
        pl.kernel,
        mesh=mesh,
        out_type=jax.ShapeDtypeStruct((_EMB_DIM, _NUM_GENES), jnp.float32),
        scratch_types=[
            pltpu.VMEM_SHARED((_SC0_SLABS, _RSLAB, _NUM_GENES), jnp.float32),
        ],
    )(_copy_body)
    return run(src).T


# speedup vs baseline: 1.0699x; 1.0160x over previous
import functools

import jax
import jax.numpy as jnp
from jax import lax
from jax.experimental import pallas as pl
from jax.experimental.pallas import tpu as pltpu
from jax.experimental.pallas import tpu_sc as plsc

_NUM_GENES = 16906
_EMB_DIM = 200

_RSLAB = 8
_NSLABS = _EMB_DIM // _RSLAB
_SC0_SLABS = 12
_SC1_SLABS = _NSLABS - _SC0_SLABS


def _copy_body(src_hbm, out_hbm, shared):
    c = lax.axis_index("c")
    s = lax.axis_index("s")
    slab = s + c * _SC0_SLABS
    n_mine = jnp.where(c == 0, _SC0_SLABS, _SC1_SLABS)

    @pl.when(s < n_mine)
    def _():
        r = pl.multiple_of(slab * _RSLAB, 8)
        pltpu.sync_copy(src_hbm.at[pl.ds(r, _RSLAB), :], shared.at[s])
        pltpu.sync_copy(shared.at[s], out_hbm.at[pl.ds(r, _RSLAB), :])


def kernel(x, gene_emb):
    del x
    src = gene_emb.T
    mesh = plsc.VectorSubcoreMesh(core_axis_name="c", subcore_axis_name="s")
    run = functools.partial(
        pl.kernel,
        mesh=mesh,
        out_type=jax.ShapeDtypeStruct((_EMB_DIM, _NUM_GENES), jnp.float32),
        scratch_types=[
            pltpu.VMEM_SHARED((_SC1_SLABS, _RSLAB, _NUM_GENES), jnp.float32),
        ],
    )(_copy_body)
    return run(src).T
